# baseline (speedup 1.0000x reference)
import functools

import jax
import jax.numpy as jnp
from jax import lax
from jax.experimental import pallas as pl
from jax.experimental.pallas import tpu as pltpu
from jax.experimental.pallas import tpu_sc as plsc

VOCAB = 30
VPAD = 32
DIM = 2048
NC = 2
NS = 16
NW = NC * NS
LAG = 8


@functools.lru_cache(maxsize=None)
def _make_gather(n_flat: int):
    b_per_w = n_flat // NW
    assert n_flat == b_per_w * NW
    mesh = plsc.VectorSubcoreMesh(core_axis_name="c", subcore_axis_name="s")

    @functools.partial(
        pl.kernel,
        mesh=mesh,
        out_type=jax.ShapeDtypeStruct((n_flat, DIM), jnp.float32),
        scratch_types=[
            pltpu.VMEM((b_per_w,), jnp.int32),
            pltpu.VMEM((VOCAB, DIM), jnp.float32),
            pltpu.SemaphoreType.DMA,
        ],
    )
    def k(idx_hbm, table_hbm, out_hbm, idx_s, table_v, ssem):
        wid = lax.axis_index("s") * NC + lax.axis_index("c")
        base = wid * b_per_w
        pltpu.sync_copy(idx_hbm.at[pl.ds(base, b_per_w)], idx_s)
        pltpu.sync_copy(table_hbm, table_v)

        def issue(i, r):
            pltpu.make_async_copy(
                table_v.at[r], out_hbm.at[base + i], ssem
            ).start()

        def drain_one():
            pltpu.make_async_copy(
                table_v.at[0], out_hbm.at[base], ssem
            ).wait()

        def group(g, drain):
            v = idx_s[pl.ds(g * 16, 16)]
            for lane in range(16):
                if drain:
                    drain_one()
                issue(g * 16 + lane, v[lane])

        group(0, drain=False)

        def step(g, carry):
            group(g, drain=True)
            return carry

        lax.fori_loop(1, b_per_w // 16, step, 0)

        def drain_loop(i, carry):
            drain_one()
            return carry

        lax.fori_loop(0, 16, drain_loop, 0)

    return k


S_SCS = 2560
SCS_CHUNK = 512
SCS_LAG = 2


@functools.lru_cache(maxsize=None)
def _make_hybrid(n_flat: int):
    from jax._src.pallas import mpmd

    n_scs = NC * S_SCS
    b_per_w = (n_flat - n_scs) // NW
    assert n_flat == n_scs + b_per_w * NW and b_per_w % 16 == 0
    assert S_SCS % SCS_CHUNK == 0
    tec_mesh = plsc.VectorSubcoreMesh(core_axis_name="c", subcore_axis_name="s")
    scs_mesh = plsc.ScalarSubcoreMesh(axis_name="c", num_cores=NC)

    def tec_fn(idx_hbm, table_hbm, out_hbm, table_sh, idx_v, table_v, ssem,
               idx_smem, scs_sem):
        del table_sh, idx_smem, scs_sem
        wid = lax.axis_index("s") * NC + lax.axis_index("c")
        base = n_scs + wid * b_per_w
        pltpu.sync_copy(idx_hbm.at[pl.ds(base, b_per_w)], idx_v)
        pltpu.sync_copy(table_hbm, table_v)

        def issue(i, r):
            pltpu.make_async_copy(
                table_v.at[r], out_hbm.at[base + i], ssem
            ).start()

        def drain_one():
            pltpu.make_async_copy(
                table_v.at[0], out_hbm.at[base], ssem
            ).wait()

        def group(g, drain):
            v = idx_v[pl.ds(g * 16, 16)]
            for lane in range(16):
                if drain:
                    drain_one()
                issue(g * 16 + lane, v[lane])

        group(0, drain=False)

        def step(g, carry):
            group(g, drain=True)
            return carry

        lax.fori_loop(1, b_per_w // 16, step, 0)

        def drain_loop(i, carry):
            drain_one()
            return carry

        lax.fori_loop(0, 16, drain_loop, 0)

    def scs_fn(idx_hbm, table_hbm, out_hbm, table_sh, idx_v, table_v, ssem,
               idx_smem, scs_sem):
        del idx_v, table_v, ssem
        cid = lax.axis_index("c")
        base = cid * S_SCS
        pltpu.sync_copy(table_hbm, table_sh)

        def issue(i, r):
            pltpu.make_async_copy(
                table_sh.at[r], out_hbm.at[base + i], scs_sem
            ).start()

        def drain_one():
            pltpu.make_async_copy(
                table_sh.at[0], out_hbm.at[base], scs_sem
            ).wait()

        def chunk_body(c, carry):
            pltpu.sync_copy(
                idx_hbm.at[pl.ds(base + c * SCS_CHUNK, SCS_CHUNK)], idx_smem
            )

            def row(i, carry2):
                g = c * SCS_CHUNK + i
                issue(g, idx_smem[i])

                @pl.when(g >= SCS_LAG)
                def _():
                    drain_one()

                return carry2

            lax.fori_loop(0, SCS_CHUNK, row, 0)
            return carry

        lax.fori_loop(0, S_SCS // SCS_CHUNK, chunk_body, 0)

        def drain_loop(i, carry):
            drain_one()
            return carry

        lax.fori_loop(0, SCS_LAG, drain_loop, 0)

    return mpmd.mpmd_map(
        [(scs_mesh, scs_fn), (tec_mesh, tec_fn)],
        out_types=jax.ShapeDtypeStruct((n_flat, DIM), jnp.float32),
        scratch_types=[
            pltpu.VMEM_SHARED((VPAD, DIM), jnp.float32),
            (pltpu.VMEM @ tec_mesh)((b_per_w,), jnp.int32),
            (pltpu.VMEM @ tec_mesh)((VPAD, DIM), jnp.float32),
            pltpu.SemaphoreType.DMA @ tec_mesh,
            (pltpu.SMEM @ scs_mesh)((SCS_CHUNK,), jnp.int32),
            pltpu.SemaphoreType.DMA @ scs_mesh,
        ],
    )


def kernel(x, table):
    B, L = x.shape
    idx = x.T.reshape(B * L).astype(jnp.int32)
    table_p = jnp.pad(table, ((0, VPAD - table.shape[0]), (0, 0)))
    out = _make_hybrid(B * L)(idx, table_p)
    return jnp.swapaxes(out.reshape(L, B, table.shape[1]), 0, 1)


# speedup vs baseline: 3.6276x; 3.6276x over previous
import functools

import jax
import jax.numpy as jnp
from jax import lax
from jax.experimental import pallas as pl
from jax.experimental.pallas import tpu as pltpu
from jax.experimental.pallas import tpu_sc as plsc

VOCAB = 30
DIM = 2048
NC = 2
NS = 16
NW = NC * NS


@functools.lru_cache(maxsize=None)
def _make_gather(n_flat: int):
    b_per_w = n_flat // NW
    assert n_flat == b_per_w * NW and b_per_w % 16 == 0
    mesh = plsc.VectorSubcoreMesh(core_axis_name="c", subcore_axis_name="s")

    @functools.partial(
        pl.kernel,
        mesh=mesh,
        out_type=jax.ShapeDtypeStruct((n_flat, DIM), jnp.float32),
        scratch_types=[
            pltpu.VMEM((b_per_w,), jnp.int32),
            pltpu.VMEM((VOCAB, DIM), jnp.float32),
            pltpu.SemaphoreType.DMA,
        ],
    )
    def k(idx_hbm, table_hbm, out_hbm, idx_s, table_v, ssem):
        wid = lax.axis_index("s") * NC + lax.axis_index("c")
        base = wid * b_per_w
        pltpu.sync_copy(idx_hbm.at[pl.ds(base, b_per_w)], idx_s)
        pltpu.sync_copy(table_hbm, table_v)

        def issue(i, r):
            pltpu.make_async_copy(
                table_v.at[r], out_hbm.at[base + i], ssem
            ).start()

        def drain_one():
            pltpu.make_async_copy(
                table_v.at[0], out_hbm.at[base], ssem
            ).wait()

        def group(g, drain):
            v = idx_s[pl.ds(g * 16, 16)]
            for lane in range(16):
                if drain:
                    drain_one()
                issue(g * 16 + lane, v[lane])

        group(0, drain=False)

        def step(g, carry):
            group(g, drain=True)
            return carry

        lax.fori_loop(1, b_per_w // 16, step, 0)

        def drain_loop(i, carry):
            drain_one()
            return carry

        lax.fori_loop(0, 16, drain_loop, 0)

    return k


def kernel(x, table):
    B, L = x.shape
    idx = x.T.reshape(B * L).astype(jnp.int32)
    out = _make_gather(B * L)(idx, table)
    return jnp.swapaxes(out.reshape(L, B, table.shape[1]), 0, 1)
